# baseline (speedup 1.0000x reference)
import functools

import jax
import jax.numpy as jnp
from jax import lax
from jax.experimental import pallas as pl
from jax.experimental.pallas import tpu as pltpu
from jax.experimental.pallas import tpu_sc as plsc


def _make_sc_kernel(N, D, n_type, n_depth, n_status, NC, NS, C):
    n_small = n_type + n_depth + n_status
    d_off = n_type
    s_off = n_type + n_depth
    NW = NC * NS
    RPW = N // NW
    NCH = RPW // C
    NH = NCH // 2
    OUTW = 8 * D

    mesh = plsc.VectorSubcoreMesh(core_axis_name="c", subcore_axis_name="s")

    def buf2(shape, dtype):
        return [pltpu.VMEM(shape, dtype), pltpu.VMEM(shape, dtype)]

    @functools.partial(
        pl.kernel,
        out_type=jax.ShapeDtypeStruct((N, OUTW), jnp.float32),
        mesh=mesh,
        compiler_params=pltpu.CompilerParams(use_tc_tiling_on_sc=False,
                                             needs_layout_passes=False),
        scratch_types=[
            buf2((9, C), jnp.int32),
            pltpu.VMEM((6, D), jnp.float32),
            pltpu.VMEM((n_small * D,), jnp.float32),
            buf2((C, 8 * D), jnp.float32),
            buf2((C, D), jnp.float32),
            buf2((C, D), jnp.float32),
            [pltpu.SemaphoreType.DMA] * 2,
            [pltpu.SemaphoreType.DMA] * 2,
            [pltpu.SemaphoreType.DMA] * 2,
        ],
    )
    def k(pkd_hbm, wb_hbm, small_hbm, contract_hbm, func_hbm,
          out_hbm,
          idx_v, wb_v, stbl_v, blk_v, gc_v, gf_v,
          sem_s, sem_g, sem_o):
        wid = lax.axis_index("s") * NC + lax.axis_index("c")
        row0 = wid * RPW
        pltpu.sync_copy(wb_hbm, wb_v)
        pltpu.sync_copy(small_hbm, stbl_v)

        def fire_stage(base, b):
            pltpu.async_copy(pkd_hbm.at[:, pl.ds(base, C)], idx_v[b], sem_s[b])

        def wait_stage(base, b):
            pltpu.make_async_copy(pkd_hbm.at[:, pl.ds(base, C)], idx_v[b], sem_s[b]).wait()

        def drain_writes(base, b):
            pltpu.make_async_copy(blk_v[b], out_hbm.at[pl.ds(base, C), :], sem_o[b]).wait()

        def fire_writes(base, b):
            pltpu.async_copy(blk_v[b], out_hbm.at[pl.ds(base, C), :], sem_o[b])

        iota16 = lax.iota(jnp.int32, 16)

        fire_stage(row0, 0)

        def outer(i, carry):
            for b in range(2):
                base = row0 + (2 * i + b) * C
                if b == 0:
                    fire_stage(base + C, 1)
                else:
                    @pl.when(i < NH - 1)
                    def _():
                        fire_stage(base + C, 0)
                wait_stage(base, b)
                @pl.when(i > 0)
                def _():
                    drain_writes(base, b)
                cp2 = pltpu.async_copy(contract_hbm.at[idx_v[b].at[1]],
                                       gc_v[b], sem_g[b])
                cp3 = pltpu.async_copy(func_hbm.at[idx_v[b].at[2]],
                                       gf_v[b], sem_g[b])
                wlo = [wb_v[2 * f, pl.ds(0, 16)] for f in range(3)]
                whi = [wb_v[2 * f, pl.ds(16, 16)] for f in range(3)]
                blo = [wb_v[2 * f + 1, pl.ds(0, 16)] for f in range(3)]
                bhi = [wb_v[2 * f + 1, pl.ds(16, 16)] for f in range(3)]

                def pgroup(g, c2):
                    sl = pl.ds(g * 16, 16)
                    xvs = [plsc.bitcast(idx_v[b][6 + f, sl], jnp.float32)
                           for f in range(3)]
                    m = idx_v[b][5, sl]
                    tvec = idx_v[b][0, sl] * D
                    dep = jnp.minimum(jnp.maximum(idx_v[b][3, sl], 0), n_depth - 2) + 1
                    st = jnp.minimum(jnp.maximum(idx_v[b][4, sl], 0), n_status - 2) + 1
                    zero = jnp.zeros((16,), jnp.int32)
                    dvec = (jnp.where(m != 0, dep, zero) + d_off) * D
                    svec = (jnp.where(m != 0, st, zero) + s_off) * D
                    for r in range(16):
                        row = g * 16 + r
                        for ids, col in ((tvec, 0), (dvec, 3 * D), (svec, 4 * D)):
                            a0 = ids[r] + iota16
                            blk_v[b][row, pl.ds(col, 16)] = plsc.load_gather(stbl_v, [a0])
                            blk_v[b][row, pl.ds(col + 16, 16)] = plsc.load_gather(stbl_v, [a0 + 16])
                        for f in range(3):
                            x = xvs[f][r]
                            lo = jnp.maximum(x * wlo[f] + blo[f], 0.0)
                            hi = jnp.maximum(x * whi[f] + bhi[f], 0.0)
                            blk_v[b][row, pl.ds(5 * D + f * 32, 16)] = lo
                            blk_v[b][row, pl.ds(5 * D + f * 32 + 16, 16)] = hi
                    return c2

                lax.fori_loop(0, C // 16, pgroup, 0)
                cp2.wait(); cp3.wait()

                def cgroup(g, c2):
                    for r in range(16):
                        row = g * 16 + r
                        for src_v, col in ((gc_v, D), (gf_v, 2 * D)):
                            blk_v[b][row, pl.ds(col, 16)] = src_v[b][row, pl.ds(0, 16)]
                            blk_v[b][row, pl.ds(col + 16, 16)] = src_v[b][row, pl.ds(16, 16)]
                    return c2

                lax.fori_loop(0, C // 16, cgroup, 0)
                fire_writes(base, b)
            return carry

        lax.fori_loop(0, NH, outer, 0)
        drain_writes(row0, 0)
        drain_writes(row0, 1)

    return k


def kernel(call_type_ids, contract_ids, func_selector_ids, depths, status_ids,
           input_sizes, output_sizes, gas_vals, trace_mask,
           type_table, contract_table, func_table, depth_table, status_table,
           W_in, b_in, W_out, b_out, W_gas, b_gas):
    B, L = call_type_ids.shape
    D = type_table.shape[1]
    N = B * L
    ids = jnp.stack([
        call_type_ids.reshape(N), contract_ids.reshape(N),
        func_selector_ids.reshape(N), depths.reshape(N),
        status_ids.reshape(N), trace_mask.reshape(N).astype(jnp.int32),
    ]).astype(jnp.int32)
    scal = jnp.stack([input_sizes.reshape(N), output_sizes.reshape(N),
                      gas_vals.reshape(N)]).astype(jnp.float32)
    pkd = jnp.concatenate([ids, lax.bitcast_convert_type(scal, jnp.int32)])
    wb = jnp.stack([W_in[:, 0], b_in, W_out[:, 0], b_out, W_gas[:, 0], b_gas])
    small = jnp.concatenate([type_table, depth_table, status_table],
                            axis=0).reshape(-1)
    info = plsc.get_sparse_core_info()
    k = _make_sc_kernel(N, D, type_table.shape[0], depth_table.shape[0],
                        status_table.shape[0], info.num_cores,
                        info.num_subcores, 128)
    out = k(pkd, wb, small, contract_table, func_table)
    return out.reshape(B, L, 8 * D)


# speedup vs baseline: 1.0989x; 1.0989x over previous
import functools

import jax
import jax.numpy as jnp
from jax import lax
from jax.experimental import pallas as pl
from jax.experimental.pallas import tpu as pltpu
from jax.experimental.pallas import tpu_sc as plsc


def _make_sc_kernel(N, D, n_type, n_depth, n_status, NC, NS, C):
    n_small = n_type + n_depth + n_status
    d_off = n_type
    s_off = n_type + n_depth
    NW = NC * NS
    RPW = N // NW
    NCH = RPW // C
    NH = NCH // 2
    OUTW = 8 * D

    mesh = plsc.VectorSubcoreMesh(core_axis_name="c", subcore_axis_name="s")

    def buf2(shape, dtype):
        return [pltpu.VMEM(shape, dtype), pltpu.VMEM(shape, dtype)]

    @functools.partial(
        pl.kernel,
        out_type=jax.ShapeDtypeStruct((N, OUTW), jnp.float32),
        mesh=mesh,
        compiler_params=pltpu.CompilerParams(use_tc_tiling_on_sc=False,
                                             needs_layout_passes=False),
        scratch_types=[
            buf2((6, C), jnp.int32),
            buf2((3, C), jnp.float32),
            pltpu.VMEM((6, D), jnp.float32),
            pltpu.VMEM((n_small * D,), jnp.float32),
            buf2((C, D), jnp.float32),
            buf2((C, D), jnp.float32),
            buf2((C, D), jnp.float32),
            buf2((C, D), jnp.float32),
            buf2((C, D), jnp.float32),
            buf2((C, 3 * D), jnp.float32),
            [pltpu.SemaphoreType.DMA] * 2,
            [pltpu.SemaphoreType.DMA] * 2,
            [pltpu.SemaphoreType.DMA] * 2,
        ],
    )
    def k(ids_hbm, scal_hbm, wb_hbm, small_hbm, contract_hbm, func_hbm,
          out_hbm,
          idx_v, scal_v, wb_v, stbl_v, gt_v, gc_v, gf_v, gd_v, gs_v, proj_v,
          sem_s, sem_g, sem_o):
        wid = lax.axis_index("s") * NC + lax.axis_index("c")
        row0 = wid * RPW
        pltpu.sync_copy(wb_hbm, wb_v)
        pltpu.sync_copy(small_hbm, stbl_v)

        def fire_stage(base, b):
            pltpu.async_copy(ids_hbm.at[:, pl.ds(base, C)], idx_v[b], sem_s[b])
            pltpu.async_copy(scal_hbm.at[:, pl.ds(base, C)], scal_v[b], sem_s[b])

        def wait_stage(base, b):
            pltpu.make_async_copy(ids_hbm.at[:, pl.ds(base, C)], idx_v[b], sem_s[b]).wait()
            pltpu.make_async_copy(scal_hbm.at[:, pl.ds(base, C)], scal_v[b], sem_s[b]).wait()

        def out_slices(base):
            return [out_hbm.at[pl.ds(base, C), pl.ds(f * D, D)] for f in range(5)] + \
                   [out_hbm.at[pl.ds(base, C), pl.ds(5 * D, 3 * D)]]

        def bufs(b):
            return [gt_v[b], gc_v[b], gf_v[b], gd_v[b], gs_v[b], proj_v[b]]

        def drain_writes(base, b):
            for src, dst in zip(bufs(b), out_slices(base)):
                pltpu.make_async_copy(src, dst, sem_o[b]).wait()

        def fire_writes(base, b):
            for src, dst in zip(bufs(b), out_slices(base)):
                pltpu.async_copy(src, dst, sem_o[b])

        iota16 = lax.iota(jnp.int32, 16)

        def fire_gathers(b):
            pltpu.async_copy(contract_hbm.at[idx_v[b].at[1]], gc_v[b], sem_g[b])
            pltpu.async_copy(func_hbm.at[idx_v[b].at[2]], gf_v[b], sem_g[b])

        def wait_gathers(b):
            pltpu.make_async_copy(contract_hbm.at[idx_v[b].at[1]], gc_v[b], sem_g[b]).wait()
            pltpu.make_async_copy(func_hbm.at[idx_v[b].at[2]], gf_v[b], sem_g[b]).wait()

        fire_stage(row0, 0)
        fire_stage(row0 + C, 1)
        wait_stage(row0, 0)
        fire_gathers(0)

        def outer(i, carry):
            for b in range(2):
                nb = 1 - b
                base = row0 + (2 * i + b) * C
                wait_gathers(b)
                if b == 0:
                    wait_stage(base + C, nb)

                    @pl.when(i > 0)
                    def _():
                        drain_writes(base, nb)
                    fire_gathers(nb)
                else:
                    @pl.when(i < NH - 1)
                    def _():
                        wait_stage(base + C, nb)
                    drain_writes(base, nb)

                    @pl.when(i < NH - 1)
                    def _():
                        fire_gathers(nb)
                wlo = [wb_v[2 * f, pl.ds(0, 16)] for f in range(3)]
                whi = [wb_v[2 * f, pl.ds(16, 16)] for f in range(3)]
                blo = [wb_v[2 * f + 1, pl.ds(0, 16)] for f in range(3)]
                bhi = [wb_v[2 * f + 1, pl.ds(16, 16)] for f in range(3)]

                def pgroup(g, c2):
                    sl = pl.ds(g * 16, 16)
                    xvs = [scal_v[b][f, sl] for f in range(3)]
                    m = idx_v[b][5, sl]
                    tvec = idx_v[b][0, sl] * D
                    dep = jnp.minimum(jnp.maximum(idx_v[b][3, sl], 0), n_depth - 2) + 1
                    st = jnp.minimum(jnp.maximum(idx_v[b][4, sl], 0), n_status - 2) + 1
                    zero = jnp.zeros((16,), jnp.int32)
                    dvec = (jnp.where(m != 0, dep, zero) + d_off) * D
                    svec = (jnp.where(m != 0, st, zero) + s_off) * D
                    for r in range(16):
                        row = g * 16 + r
                        for ids, dst in ((tvec, gt_v), (dvec, gd_v), (svec, gs_v)):
                            a0 = ids[r] + iota16
                            dst[b][row, pl.ds(0, 16)] = plsc.load_gather(stbl_v, [a0])
                            dst[b][row, pl.ds(16, 16)] = plsc.load_gather(stbl_v, [a0 + 16])
                        for f in range(3):
                            x = xvs[f][r]
                            lo = jnp.maximum(x * wlo[f] + blo[f], 0.0)
                            hi = jnp.maximum(x * whi[f] + bhi[f], 0.0)
                            proj_v[b][row, pl.ds(f * 32, 16)] = lo
                            proj_v[b][row, pl.ds(f * 32 + 16, 16)] = hi
                    return c2

                lax.fori_loop(0, C // 16, pgroup, 0)
                fire_writes(base, b)
                @pl.when(i < NH - 1)
                def _():
                    fire_stage(base + 2 * C, b)
            return carry

        lax.fori_loop(0, NH, outer, 0)
        drain_writes(row0, 1)

    return k


def kernel(call_type_ids, contract_ids, func_selector_ids, depths, status_ids,
           input_sizes, output_sizes, gas_vals, trace_mask,
           type_table, contract_table, func_table, depth_table, status_table,
           W_in, b_in, W_out, b_out, W_gas, b_gas):
    B, L = call_type_ids.shape
    D = type_table.shape[1]
    N = B * L
    ids = jnp.stack([
        call_type_ids.reshape(N), contract_ids.reshape(N),
        func_selector_ids.reshape(N), depths.reshape(N),
        status_ids.reshape(N), trace_mask.reshape(N).astype(jnp.int32),
    ]).astype(jnp.int32)
    scal = jnp.stack([input_sizes.reshape(N), output_sizes.reshape(N),
                      gas_vals.reshape(N)]).astype(jnp.float32)
    wb = jnp.stack([W_in[:, 0], b_in, W_out[:, 0], b_out, W_gas[:, 0], b_gas])
    small = jnp.concatenate([type_table, depth_table, status_table],
                            axis=0).reshape(-1)
    info = plsc.get_sparse_core_info()
    k = _make_sc_kernel(N, D, type_table.shape[0], depth_table.shape[0],
                        status_table.shape[0], info.num_cores,
                        info.num_subcores, 128)
    out = k(ids, scal, wb, small, contract_table, func_table)
    return out.reshape(B, L, 8 * D)


# speedup vs baseline: 1.1136x; 1.0134x over previous
import functools

import jax
import jax.numpy as jnp
from jax import lax
from jax.experimental import pallas as pl
from jax.experimental.pallas import tpu as pltpu
from jax.experimental.pallas import tpu_sc as plsc


def _make_sc_kernel(N, D, n_type, n_depth, n_status, NC, NS, C):
    n_small = n_type + n_depth + n_status
    d_off = n_type
    s_off = n_type + n_depth
    NW = NC * NS
    RPW = N // NW
    NCH = RPW // C
    K = (NCH - 2) // 4
    assert NCH == 4 * K + 2
    OUTW = 8 * D

    mesh = plsc.VectorSubcoreMesh(core_axis_name="c", subcore_axis_name="s")

    def bufn(n, shape, dtype):
        return [pltpu.VMEM(shape, dtype) for _ in range(n)]

    @functools.partial(
        pl.kernel,
        out_type=jax.ShapeDtypeStruct((N, OUTW), jnp.float32),
        mesh=mesh,
        compiler_params=pltpu.CompilerParams(use_tc_tiling_on_sc=False,
                                             needs_layout_passes=False),
        scratch_types=[
            bufn(2, (6, C), jnp.int32),
            bufn(2, (3, C), jnp.float32),
            pltpu.VMEM((6, D), jnp.float32),
            pltpu.VMEM((n_small * D,), jnp.float32),
            bufn(2, (C, D), jnp.float32),
            bufn(4, (C, D), jnp.float32),
            bufn(4, (C, D), jnp.float32),
            bufn(2, (C, D), jnp.float32),
            bufn(2, (C, D), jnp.float32),
            bufn(2, (C, 3 * D), jnp.float32),
            [pltpu.SemaphoreType.DMA] * 2,
            [pltpu.SemaphoreType.DMA] * 4,
            [pltpu.SemaphoreType.DMA] * 2,
        ],
    )
    def k(ids_hbm, scal_hbm, wb_hbm, small_hbm, contract_hbm, func_hbm,
          out_hbm,
          idx_v, scal_v, wb_v, stbl_v, gt_v, gc_v, gf_v, gd_v, gs_v, proj_v,
          sem_s, sem_g, sem_o):
        wid = lax.axis_index("s") * NC + lax.axis_index("c")
        row0 = wid * RPW
        pltpu.sync_copy(wb_hbm, wb_v)
        pltpu.sync_copy(small_hbm, stbl_v)

        def fire_stage(base, b):
            pltpu.async_copy(ids_hbm.at[:, pl.ds(base, C)], idx_v[b], sem_s[b])
            pltpu.async_copy(scal_hbm.at[:, pl.ds(base, C)], scal_v[b], sem_s[b])

        def wait_stage(base, b):
            pltpu.make_async_copy(ids_hbm.at[:, pl.ds(base, C)], idx_v[b], sem_s[b]).wait()
            pltpu.make_async_copy(scal_hbm.at[:, pl.ds(base, C)], scal_v[b], sem_s[b]).wait()

        def fire_gathers(b, q):
            pltpu.async_copy(contract_hbm.at[idx_v[b].at[1]], gc_v[q], sem_g[q])
            pltpu.async_copy(func_hbm.at[idx_v[b].at[2]], gf_v[q], sem_g[q])

        def wait_gathers(b, q):
            pltpu.make_async_copy(contract_hbm.at[idx_v[b].at[1]], gc_v[q], sem_g[q]).wait()
            pltpu.make_async_copy(func_hbm.at[idx_v[b].at[2]], gf_v[q], sem_g[q]).wait()

        def wpairs(base, b, q):
            srcs = [gt_v[b], gc_v[q], gf_v[q], gd_v[b], gs_v[b], proj_v[b]]
            dsts = [out_hbm.at[pl.ds(base, C), pl.ds(f * D, D)] for f in range(5)] + \
                   [out_hbm.at[pl.ds(base, C), pl.ds(5 * D, 3 * D)]]
            return zip(srcs, dsts)

        def drain_writes(base, b, q):
            for src, dst in wpairs(base, b, q):
                pltpu.make_async_copy(src, dst, sem_o[b]).wait()

        def fire_writes(base, b, q):
            for src, dst in wpairs(base, b, q):
                pltpu.async_copy(src, dst, sem_o[b])

        iota16 = lax.iota(jnp.int32, 16)

        def pgroup_loop(b):
            wlo = [wb_v[2 * f, pl.ds(0, 16)] for f in range(3)]
            whi = [wb_v[2 * f, pl.ds(16, 16)] for f in range(3)]
            blo = [wb_v[2 * f + 1, pl.ds(0, 16)] for f in range(3)]
            bhi = [wb_v[2 * f + 1, pl.ds(16, 16)] for f in range(3)]

            def pgroup(g, c2):
                sl = pl.ds(g * 16, 16)
                xvs = [scal_v[b][f, sl] for f in range(3)]
                m = idx_v[b][5, sl]
                tvec = idx_v[b][0, sl] * D
                dep = jnp.minimum(jnp.maximum(idx_v[b][3, sl], 0), n_depth - 2) + 1
                st = jnp.minimum(jnp.maximum(idx_v[b][4, sl], 0), n_status - 2) + 1
                zero = jnp.zeros((16,), jnp.int32)
                dvec = (jnp.where(m != 0, dep, zero) + d_off) * D
                svec = (jnp.where(m != 0, st, zero) + s_off) * D
                for r in range(16):
                    row = g * 16 + r
                    for ids, dst in ((tvec, gt_v), (dvec, gd_v), (svec, gs_v)):
                        a0 = ids[r] + iota16
                        dst[b][row, pl.ds(0, 16)] = plsc.load_gather(stbl_v, [a0])
                        dst[b][row, pl.ds(16, 16)] = plsc.load_gather(stbl_v, [a0 + 16])
                    for f in range(3):
                        x = xvs[f][r]
                        lo = jnp.maximum(x * wlo[f] + blo[f], 0.0)
                        hi = jnp.maximum(x * whi[f] + bhi[f], 0.0)
                        proj_v[b][row, pl.ds(f * 32, 16)] = lo
                        proj_v[b][row, pl.ds(f * 32 + 16, 16)] = hi
                return c2

            lax.fori_loop(0, C // 16, pgroup, 0)

        def phase(base, b, q, drain_pred, has_next, has_next2):
            wait_gathers(b, q)
            if has_next:
                wait_stage(base + C, 1 - b)
            if drain_pred is True:
                drain_writes(base, b, (q + 2) % 4)
            elif drain_pred is not False:
                @pl.when(drain_pred)
                def _():
                    drain_writes(base, b, (q + 2) % 4)
            if has_next:
                fire_gathers(1 - b, (q + 1) % 4)
            pgroup_loop(b)
            fire_writes(base, b, q)
            if has_next2:
                fire_stage(base + 2 * C, b)

        fire_stage(row0, 0)
        fire_stage(row0 + C, 1)
        wait_stage(row0, 0)
        fire_gathers(0, 0)

        def outer(i, carry):
            for p in range(4):
                base = row0 + (4 * i + p) * C
                drain = True if p >= 2 else (i > 0)
                phase(base, p % 2, p, drain, True, True)
            return carry

        lax.fori_loop(0, K, outer, 0)
        base = row0 + (NCH - 2) * C
        phase(base, 0, (NCH - 2) % 4, True, True, False)
        phase(base + C, 1, (NCH - 1) % 4, True, False, False)
        drain_writes(row0, 0, (NCH - 2) % 4)
        drain_writes(row0, 1, (NCH - 1) % 4)

    return k


def kernel(call_type_ids, contract_ids, func_selector_ids, depths, status_ids,
           input_sizes, output_sizes, gas_vals, trace_mask,
           type_table, contract_table, func_table, depth_table, status_table,
           W_in, b_in, W_out, b_out, W_gas, b_gas):
    B, L = call_type_ids.shape
    D = type_table.shape[1]
    N = B * L
    ids = jnp.stack([
        call_type_ids.reshape(N), contract_ids.reshape(N),
        func_selector_ids.reshape(N), depths.reshape(N),
        status_ids.reshape(N), trace_mask.reshape(N).astype(jnp.int32),
    ]).astype(jnp.int32)
    scal = jnp.stack([input_sizes.reshape(N), output_sizes.reshape(N),
                      gas_vals.reshape(N)]).astype(jnp.float32)
    wb = jnp.stack([W_in[:, 0], b_in, W_out[:, 0], b_out, W_gas[:, 0], b_gas])
    small = jnp.concatenate([type_table, depth_table, status_table],
                            axis=0).reshape(-1)
    info = plsc.get_sparse_core_info()
    k = _make_sc_kernel(N, D, type_table.shape[0], depth_table.shape[0],
                        status_table.shape[0], info.num_cores,
                        info.num_subcores, 128)
    out = k(ids, scal, wb, small, contract_table, func_table)
    return out.reshape(B, L, 8 * D)


# speedup vs baseline: 1.1932x; 1.0715x over previous
import functools

import jax
import jax.numpy as jnp
from jax import lax
from jax.experimental import pallas as pl
from jax.experimental.pallas import tpu as pltpu
from jax.experimental.pallas import tpu_sc as plsc


def _make_sc_kernel(N, D, n_type, n_depth, n_status, NC, NS, C):
    n_small = n_type + n_depth + n_status
    d_off = n_type
    s_off = n_type + n_depth
    NW = NC * NS
    RPW = N // NW
    NCH = RPW // C
    NH = NCH // 2
    OUTW = 8 * D

    mesh = plsc.VectorSubcoreMesh(core_axis_name="c", subcore_axis_name="s")

    def buf2(shape, dtype):
        return [pltpu.VMEM(shape, dtype), pltpu.VMEM(shape, dtype)]

    @functools.partial(
        pl.kernel,
        out_type=jax.ShapeDtypeStruct((N, OUTW), jnp.float32),
        mesh=mesh,
        compiler_params=pltpu.CompilerParams(use_tc_tiling_on_sc=False,
                                             needs_layout_passes=False),
        scratch_types=[
            buf2((6, C), jnp.int32),
            buf2((3, C), jnp.float32),
            pltpu.VMEM((6, D), jnp.float32),
            pltpu.VMEM((n_small * D,), jnp.float32),
            buf2((C, D), jnp.float32),
            buf2((C, D), jnp.float32),
            buf2((C, D), jnp.float32),
            buf2((C, D), jnp.float32),
            buf2((C, D), jnp.float32),
            buf2((C, 3 * D), jnp.float32),
            [pltpu.SemaphoreType.DMA] * 2,
            [pltpu.SemaphoreType.DMA] * 2,
            [pltpu.SemaphoreType.DMA] * 2,
        ],
    )
    def k(ids_hbm, scal_hbm, wb_hbm, small_hbm, contract_hbm, func_hbm,
          out_hbm,
          idx_v, scal_v, wb_v, stbl_v, gt_v, gc_v, gf_v, gd_v, gs_v, proj_v,
          sem_s, sem_g, sem_o):
        wid = lax.axis_index("s") * NC + lax.axis_index("c")
        row0 = wid * RPW
        pltpu.sync_copy(wb_hbm, wb_v)
        pltpu.sync_copy(small_hbm, stbl_v)

        def fire_stage(base, b):
            pltpu.async_copy(ids_hbm.at[:, pl.ds(base, C)], idx_v[b], sem_s[b])
            pltpu.async_copy(scal_hbm.at[:, pl.ds(base, C)], scal_v[b], sem_s[b])

        def wait_stage(base, b):
            pltpu.make_async_copy(ids_hbm.at[:, pl.ds(base, C)], idx_v[b], sem_s[b]).wait()
            pltpu.make_async_copy(scal_hbm.at[:, pl.ds(base, C)], scal_v[b], sem_s[b]).wait()

        def out_slices(base):
            return [out_hbm.at[pl.ds(base, C), pl.ds(f * D, D)] for f in range(5)] + \
                   [out_hbm.at[pl.ds(base, C), pl.ds(5 * D, 3 * D)]]

        def bufs(b):
            return [gt_v[b], gc_v[b], gf_v[b], gd_v[b], gs_v[b], proj_v[b]]

        def drain_writes(base, b):
            for src, dst in zip(bufs(b), out_slices(base)):
                pltpu.make_async_copy(src, dst, sem_o[b]).wait()

        def fire_writes(base, b):
            for src, dst in zip(bufs(b), out_slices(base)):
                pltpu.async_copy(src, dst, sem_o[b])

        iota16 = lax.iota(jnp.int32, 16)

        fire_stage(row0, 0)

        def outer(i, carry):
            for b in range(2):
                base = row0 + (2 * i + b) * C
                if b == 0:
                    fire_stage(base + C, 1)
                else:
                    @pl.when(i < NH - 1)
                    def _():
                        fire_stage(base + C, 0)
                wait_stage(base, b)
                @pl.when(i > 0)
                def _():
                    drain_writes(base, b)
                cp2 = pltpu.async_copy(contract_hbm.at[idx_v[b].at[1]], gc_v[b], sem_g[b])
                cp3 = pltpu.async_copy(func_hbm.at[idx_v[b].at[2]], gf_v[b], sem_g[b])
                wlo = [wb_v[2 * f, pl.ds(0, 16)] for f in range(3)]
                whi = [wb_v[2 * f, pl.ds(16, 16)] for f in range(3)]
                blo = [wb_v[2 * f + 1, pl.ds(0, 16)] for f in range(3)]
                bhi = [wb_v[2 * f + 1, pl.ds(16, 16)] for f in range(3)]

                def pgroup(g, c2):
                    sl = pl.ds(g * 16, 16)
                    xvs = [scal_v[b][f, sl] for f in range(3)]
                    m = idx_v[b][5, sl]
                    tvec = idx_v[b][0, sl] * D
                    dep = jnp.minimum(jnp.maximum(idx_v[b][3, sl], 0), n_depth - 2) + 1
                    st = jnp.minimum(jnp.maximum(idx_v[b][4, sl], 0), n_status - 2) + 1
                    zero = jnp.zeros((16,), jnp.int32)
                    dvec = (jnp.where(m != 0, dep, zero) + d_off) * D
                    svec = (jnp.where(m != 0, st, zero) + s_off) * D
                    for r in range(16):
                        row = g * 16 + r
                        for ids, dst in ((tvec, gt_v), (dvec, gd_v), (svec, gs_v)):
                            a0 = ids[r] + iota16
                            dst[b][row, pl.ds(0, 16)] = plsc.load_gather(stbl_v, [a0])
                            dst[b][row, pl.ds(16, 16)] = plsc.load_gather(stbl_v, [a0 + 16])
                        for f in range(3):
                            x = xvs[f][r]
                            lo = jnp.maximum(x * wlo[f] + blo[f], 0.0)
                            hi = jnp.maximum(x * whi[f] + bhi[f], 0.0)
                            proj_v[b][row, pl.ds(f * 32, 16)] = lo
                            proj_v[b][row, pl.ds(f * 32 + 16, 16)] = hi
                    return c2

                lax.fori_loop(0, C // 16, pgroup, 0)
                cp2.wait(); cp3.wait()
                fire_writes(base, b)
            return carry

        lax.fori_loop(0, NH, outer, 0)
        drain_writes(row0, 0)
        drain_writes(row0, 1)

    return k


def kernel(call_type_ids, contract_ids, func_selector_ids, depths, status_ids,
           input_sizes, output_sizes, gas_vals, trace_mask,
           type_table, contract_table, func_table, depth_table, status_table,
           W_in, b_in, W_out, b_out, W_gas, b_gas):
    B, L = call_type_ids.shape
    D = type_table.shape[1]
    N = B * L
    ids = jnp.stack([
        call_type_ids.reshape(N), contract_ids.reshape(N),
        func_selector_ids.reshape(N), depths.reshape(N),
        status_ids.reshape(N), trace_mask.reshape(N).astype(jnp.int32),
    ]).astype(jnp.int32)
    scal = jnp.stack([input_sizes.reshape(N), output_sizes.reshape(N),
                      gas_vals.reshape(N)]).astype(jnp.float32)
    wb = jnp.stack([W_in[:, 0], b_in, W_out[:, 0], b_out, W_gas[:, 0], b_gas])
    small = jnp.concatenate([type_table, depth_table, status_table],
                            axis=0).reshape(-1)
    info = plsc.get_sparse_core_info()
    k = _make_sc_kernel(N, D, type_table.shape[0], depth_table.shape[0],
                        status_table.shape[0], info.num_cores,
                        info.num_subcores, 128)
    out = k(ids, scal, wb, small, contract_table, func_table)
    return out.reshape(B, L, 8 * D)


# speedup vs baseline: 1.2786x; 1.0716x over previous
import functools

import jax
import jax.numpy as jnp
from jax import lax
from jax.experimental import pallas as pl
from jax.experimental.pallas import tpu as pltpu
from jax.experimental.pallas import tpu_sc as plsc


def _make_sc_kernel(N, D, n_type, n_depth, n_status, NC, NS, C):
    n_small = n_type + n_depth + n_status
    d_off = n_type
    s_off = n_type + n_depth
    NW = NC * NS
    RPW = N // NW
    NCH = RPW // C
    NH = NCH // 2
    OUTW = 8 * D

    mesh = plsc.VectorSubcoreMesh(core_axis_name="c", subcore_axis_name="s")

    def buf2(shape, dtype):
        return [pltpu.VMEM(shape, dtype), pltpu.VMEM(shape, dtype)]

    @functools.partial(
        pl.kernel,
        out_type=jax.ShapeDtypeStruct((N, OUTW), jnp.float32),
        mesh=mesh,
        compiler_params=pltpu.CompilerParams(use_tc_tiling_on_sc=False,
                                             needs_layout_passes=False),
        scratch_types=[
            buf2((6, C), jnp.int32),
            buf2((3, C), jnp.float32),
            pltpu.VMEM((6, D), jnp.float32),
            pltpu.VMEM((n_small * D,), jnp.float32),
            buf2((C, D), jnp.float32),
            buf2((C, D), jnp.float32),
            buf2((C, D), jnp.float32),
            buf2((C, D), jnp.float32),
            buf2((C, D), jnp.float32),
            buf2((C, 3 * D), jnp.float32),
            [pltpu.SemaphoreType.DMA] * 2,
            [pltpu.SemaphoreType.DMA] * 2,
            [pltpu.SemaphoreType.DMA] * 2,
        ],
    )
    def k(ct_hbm, co_hbm, fu_hbm, de_hbm, st_hbm, mk_hbm,
          si_hbm, so_hbm, sg_hbm, wb_hbm, small_hbm, contract_hbm, func_hbm,
          out_hbm,
          idx_v, scal_v, wb_v, stbl_v, gt_v, gc_v, gf_v, gd_v, gs_v, proj_v,
          sem_s, sem_g, sem_o):
        wid = lax.axis_index("s") * NC + lax.axis_index("c")
        row0 = wid * RPW
        pltpu.sync_copy(wb_hbm, wb_v)
        pltpu.sync_copy(small_hbm, stbl_v)

        id_srcs = [ct_hbm, co_hbm, fu_hbm, de_hbm, st_hbm, mk_hbm]
        sc_srcs = [si_hbm, so_hbm, sg_hbm]

        def fire_stage(base, b):
            for f, src in enumerate(id_srcs):
                pltpu.async_copy(src.at[pl.ds(base, C)], idx_v[b].at[f], sem_s[b])
            for f, src in enumerate(sc_srcs):
                pltpu.async_copy(src.at[pl.ds(base, C)], scal_v[b].at[f], sem_s[b])

        def wait_stage(base, b):
            for f, src in enumerate(id_srcs):
                pltpu.make_async_copy(src.at[pl.ds(base, C)], idx_v[b].at[f], sem_s[b]).wait()
            for f, src in enumerate(sc_srcs):
                pltpu.make_async_copy(src.at[pl.ds(base, C)], scal_v[b].at[f], sem_s[b]).wait()

        def out_slices(base):
            return [out_hbm.at[pl.ds(base, C), pl.ds(f * D, D)] for f in range(5)] + \
                   [out_hbm.at[pl.ds(base, C), pl.ds(5 * D, 3 * D)]]

        def bufs(b):
            return [gt_v[b], gc_v[b], gf_v[b], gd_v[b], gs_v[b], proj_v[b]]

        def drain_writes(base, b):
            for src, dst in zip(bufs(b), out_slices(base)):
                pltpu.make_async_copy(src, dst, sem_o[b]).wait()

        def fire_writes(base, b):
            for src, dst in zip(bufs(b), out_slices(base)):
                pltpu.async_copy(src, dst, sem_o[b])

        iota16 = lax.iota(jnp.int32, 16)

        fire_stage(row0, 0)

        def outer(i, carry):
            for b in range(2):
                base = row0 + (2 * i + b) * C
                if b == 0:
                    fire_stage(base + C, 1)
                else:
                    @pl.when(i < NH - 1)
                    def _():
                        fire_stage(base + C, 0)
                wait_stage(base, b)
                @pl.when(i > 0)
                def _():
                    drain_writes(base, b)
                cp2 = pltpu.async_copy(contract_hbm.at[idx_v[b].at[1]], gc_v[b], sem_g[b])
                cp3 = pltpu.async_copy(func_hbm.at[idx_v[b].at[2]], gf_v[b], sem_g[b])
                wlo = [wb_v[2 * f, pl.ds(0, 16)] for f in range(3)]
                whi = [wb_v[2 * f, pl.ds(16, 16)] for f in range(3)]
                blo = [wb_v[2 * f + 1, pl.ds(0, 16)] for f in range(3)]
                bhi = [wb_v[2 * f + 1, pl.ds(16, 16)] for f in range(3)]

                def pgroup(g, c2):
                    sl = pl.ds(g * 16, 16)
                    xvs = [scal_v[b][f, sl] for f in range(3)]
                    m = idx_v[b][5, sl]
                    tvec = idx_v[b][0, sl] * D
                    dep = jnp.minimum(jnp.maximum(idx_v[b][3, sl], 0), n_depth - 2) + 1
                    st = jnp.minimum(jnp.maximum(idx_v[b][4, sl], 0), n_status - 2) + 1
                    zero = jnp.zeros((16,), jnp.int32)
                    dvec = (jnp.where(m != 0, dep, zero) + d_off) * D
                    svec = (jnp.where(m != 0, st, zero) + s_off) * D
                    for r in range(16):
                        row = g * 16 + r
                        for ids, dst in ((tvec, gt_v), (dvec, gd_v), (svec, gs_v)):
                            a0 = ids[r] + iota16
                            dst[b][row, pl.ds(0, 16)] = plsc.load_gather(stbl_v, [a0])
                            dst[b][row, pl.ds(16, 16)] = plsc.load_gather(stbl_v, [a0 + 16])
                        for f in range(3):
                            x = xvs[f][r]
                            lo = jnp.maximum(x * wlo[f] + blo[f], 0.0)
                            hi = jnp.maximum(x * whi[f] + bhi[f], 0.0)
                            proj_v[b][row, pl.ds(f * 32, 16)] = lo
                            proj_v[b][row, pl.ds(f * 32 + 16, 16)] = hi
                    return c2

                lax.fori_loop(0, C // 16, pgroup, 0)
                cp2.wait(); cp3.wait()
                fire_writes(base, b)
            return carry

        lax.fori_loop(0, NH, outer, 0)
        drain_writes(row0, 0)
        drain_writes(row0, 1)

    return k


def kernel(call_type_ids, contract_ids, func_selector_ids, depths, status_ids,
           input_sizes, output_sizes, gas_vals, trace_mask,
           type_table, contract_table, func_table, depth_table, status_table,
           W_in, b_in, W_out, b_out, W_gas, b_gas):
    B, L = call_type_ids.shape
    D = type_table.shape[1]
    N = B * L
    wb = jnp.stack([W_in[:, 0], b_in, W_out[:, 0], b_out, W_gas[:, 0], b_gas])
    small = jnp.concatenate([type_table, depth_table, status_table],
                            axis=0).reshape(-1)
    info = plsc.get_sparse_core_info()
    k = _make_sc_kernel(N, D, type_table.shape[0], depth_table.shape[0],
                        status_table.shape[0], info.num_cores,
                        info.num_subcores, 128)
    out = k(call_type_ids.reshape(N).astype(jnp.int32),
            contract_ids.reshape(N).astype(jnp.int32),
            func_selector_ids.reshape(N).astype(jnp.int32),
            depths.reshape(N).astype(jnp.int32),
            status_ids.reshape(N).astype(jnp.int32),
            trace_mask.reshape(N).astype(jnp.int32),
            input_sizes.reshape(N).astype(jnp.float32),
            output_sizes.reshape(N).astype(jnp.float32),
            gas_vals.reshape(N).astype(jnp.float32),
            wb, small, contract_table, func_table)
    return out.reshape(B, L, 8 * D)
